# Initial kernel scaffold; baseline (speedup 1.0000x reference)
import jax
import jax.numpy as jnp
from jax.experimental import pallas as pl


def kernel(x, edge_index, Wroot, Wrel, b, S):
    raise NotImplementedError("write your pallas kernel here")


# speedup vs baseline: 6.3245x; 6.3245x over previous
import functools

import jax
import jax.numpy as jnp
from jax import lax
from jax.experimental import pallas as pl
from jax.experimental.pallas import tpu as pltpu
from jax.experimental.pallas import tpu_sc as plsc

N = 10000
E = 320000
D = 128
H = 128
NUM_LEVELS = 3
NUM_PROC = 9

NC = 2
NS = 16
NW = NC * NS
EPW = E // NW
CH = 128
NFULL = EPW // CH
REM = EPW - NFULL * CH
NP = 10240
ROWS_PER_TILE = NP // NS


def _sc_body(x_hbm, src_hbm, dst_hbm, zz_hbm, out_hbm,
             sidx, didx, rows, sidx_r, didx_r, rows_r, aggr_sh, sem):
    c = lax.axis_index("c")
    s = lax.axis_index("s")
    wid = s * NC + c
    pltpu.sync_copy(zz_hbm.at[pl.ds(s * ROWS_PER_TILE, ROWS_PER_TILE)],
                    aggr_sh.at[pl.ds(s * ROWS_PER_TILE, ROWS_PER_TILE)])
    plsc.subcore_barrier()

    base = wid * EPW

    def chunk(k, carry):
        off = base + k * CH
        pltpu.sync_copy(src_hbm.at[pl.ds(off, CH)], sidx)
        pltpu.sync_copy(dst_hbm.at[pl.ds(off, CH)], didx)
        pltpu.async_copy(x_hbm.at[sidx], rows, sem).wait()
        pltpu.sync_copy(rows, aggr_sh.at[didx], add=True)
        return carry

    lax.fori_loop(0, NFULL, chunk, 0)

    off = base + NFULL * CH
    pltpu.sync_copy(src_hbm.at[pl.ds(off, REM)], sidx_r)
    pltpu.sync_copy(dst_hbm.at[pl.ds(off, REM)], didx_r)
    pltpu.async_copy(x_hbm.at[sidx_r], rows_r, sem).wait()
    pltpu.sync_copy(rows_r, aggr_sh.at[didx_r], add=True)

    plsc.subcore_barrier()
    pltpu.sync_copy(aggr_sh.at[pl.ds(s * ROWS_PER_TILE, ROWS_PER_TILE)],
                    out_hbm.at[c, pl.ds(s * ROWS_PER_TILE, ROWS_PER_TILE)])


@functools.cache
def _sc_segment_sum():
    return pl.kernel(
        _sc_body,
        out_type=jax.ShapeDtypeStruct((NC, NP, D), jnp.float32),
        mesh=plsc.VectorSubcoreMesh(core_axis_name="c", subcore_axis_name="s",
                                    num_cores=NC, num_subcores=NS),
        scratch_types=[
            pltpu.VMEM((CH,), jnp.int32),
            pltpu.VMEM((CH,), jnp.int32),
            pltpu.VMEM((CH, D), jnp.float32),
            pltpu.VMEM((REM,), jnp.int32),
            pltpu.VMEM((REM,), jnp.int32),
            pltpu.VMEM((REM, D), jnp.float32),
            pltpu.VMEM_SHARED((NP, D), jnp.float32),
            pltpu.SemaphoreType.DMA,
        ],
    )


ROW_BLK = 1000


def _tc_body(x_ref, p0_ref, p1_ref, wroot_ref, wrel_ref, b_ref, s_ref,
             *out_refs):
    xb = x_ref[...]
    ab = p0_ref[...] + p1_ref[...]
    for level in range(NUM_LEVELS):
        xc = (jnp.dot(xb, wroot_ref[level], preferred_element_type=jnp.float32)
              + jnp.dot(ab, wrel_ref[level], preferred_element_type=jnp.float32)
              + b_ref[level][None, :])
        for j in range(3):
            p = level * 3 + j
            out_refs[p][...] = jnp.dot(xc, s_ref[p],
                                       preferred_element_type=jnp.float32)


def _tc_dense(x, p0, p1, Wroot, Wrel, b, S):
    grid = (N // ROW_BLK,)
    row_spec = pl.BlockSpec((ROW_BLK, D), lambda i: (i, 0))
    full = lambda shape: pl.BlockSpec(shape, lambda i: (0,) * len(shape))
    return pl.pallas_call(
        _tc_body,
        grid=grid,
        in_specs=[
            row_spec, row_spec, row_spec,
            full((NUM_LEVELS, D, H)),
            full((NUM_LEVELS, D, H)),
            full((NUM_LEVELS, H)),
            full((NUM_PROC, H, H)),
        ],
        out_specs=tuple(pl.BlockSpec((ROW_BLK, H), lambda i: (i, 0))
                        for _ in range(NUM_PROC)),
        out_shape=tuple(jax.ShapeDtypeStruct((N, H), jnp.float32)
                        for _ in range(NUM_PROC)),
    )(x, p0, p1, Wroot, Wrel, b, S)


def kernel(x, edge_index, Wroot, Wrel, b, S):
    src = edge_index[0]
    dst = edge_index[1]
    zz = jnp.zeros((NP, D), jnp.float32)
    parts = _sc_segment_sum()(x, src, dst, zz)
    outs = _tc_dense(x, parts[0, :N], parts[1, :N], Wroot, Wrel, b, S)
    return tuple(outs)
